# baseline (speedup 1.0000x reference)
import functools

import jax
import jax.numpy as jnp
from jax import lax
from jax.experimental import pallas as pl
from jax.experimental.pallas import tpu as pltpu
from jax.experimental.pallas import tpu_sc as plsc

NC = 2
NS = 16
NW = NC * NS
LANES = 16


def _proj_body(h_ref, ws_ref, wd_ref, wf_ref, a_ref, b_ref):
    h = h_ref[...]
    wf = wf_ref[...]
    dn = (((1,), (1,)), ((), ()))
    ts = lax.dot_general(h, ws_ref[...], dn, preferred_element_type=jnp.float32)
    a_ref[...] = lax.dot_general(ts, wf, dn, preferred_element_type=jnp.float32)
    td = lax.dot_general(h, wd_ref[...], dn, preferred_element_type=jnp.float32)
    b_ref[...] = lax.dot_general(td, wf, dn, preferred_element_type=jnp.float32)


def _project(h, W_src, W_dst, W_fc):
    n, f = h.shape
    o2 = W_src.shape[0]
    blk = 2000 if n % 2000 == 0 else n
    grid = (n // blk,)
    return pl.pallas_call(
        _proj_body,
        grid=grid,
        in_specs=[
            pl.BlockSpec((blk, f), lambda i: (i, 0)),
            pl.BlockSpec((o2, f), lambda i: (0, 0)),
            pl.BlockSpec((o2, f), lambda i: (0, 0)),
            pl.BlockSpec((o2, o2), lambda i: (0, 0)),
        ],
        out_specs=[
            pl.BlockSpec((blk, o2), lambda i: (i, 0)),
            pl.BlockSpec((blk, o2), lambda i: (i, 0)),
        ],
        out_shape=[
            jax.ShapeDtypeStruct((n, o2), jnp.float32),
            jax.ShapeDtypeStruct((n, o2), jnp.float32),
        ],
    )(h, W_src, W_dst, W_fc)


_TF_ROT = ((13, 15, 26, 6), (17, 29, 16, 24))


def _rotl(x, r):
    return (x << jnp.uint32(r)) | (x >> jnp.uint32(32 - r))


def _threefry_bits(idx):
    k1 = jnp.uint32(0)
    k2 = jnp.uint32(42)
    ks = (k1, k2, k1 ^ k2 ^ jnp.uint32(0x1BD11BDA))
    x0 = jnp.zeros_like(idx) + ks[0]
    x1 = idx + ks[1]
    for i in range(5):
        for r in _TF_ROT[i % 2]:
            x0 = x0 + x1
            x1 = _rotl(x1, r)
            x1 = x1 ^ x0
        x0 = x0 + ks[(i + 1) % 3]
        x1 = x1 + ks[(i + 2) % 3] + jnp.uint32(i + 1)
    return x0 ^ x1


def _erfinv(x):
    w = -jnp.log1p(-x * x)
    wl = w - 2.5
    p = jnp.float32(2.81022636e-08)
    for c in (3.43273939e-07, -3.5233877e-06, -4.39150654e-06, 0.00021858087,
              -0.00125372503, -0.00417768164, 0.246640727, 1.50140941):
        p = jnp.float32(c) + p * wl
    ws = jnp.sqrt(w) - 3.0
    q = jnp.float32(-0.000200214257)
    for c in (0.000100950558, 0.00134934322, -0.00367342844, 0.00573950773,
              -0.0076224613, 0.00943887047, 1.00167406, 2.83297682):
        q = jnp.float32(c) + q * ws
    return jnp.where(w < 5.0, p, q) * x


def _rng_body(blk, cols, o_ref):
    i = pl.program_id(0)
    base = i * blk * cols
    r_iota = lax.broadcasted_iota(jnp.int32, (blk, cols), 0)
    c_iota = lax.broadcasted_iota(jnp.int32, (blk, cols), 1)
    idx = (base + r_iota * cols + c_iota).astype(jnp.uint32)
    bits = _threefry_bits(idx)
    f = lax.bitcast_convert_type((bits >> jnp.uint32(9)) | jnp.uint32(0x3F800000),
                                 jnp.float32)
    lo = jnp.float32(-0.99999994)
    hi = jnp.float32(1.0)
    u = jnp.maximum(lo, f * (hi - lo) + (lo - (hi - lo)))
    o_ref[...] = jnp.float32(1.4142135623730951) * _erfinv(u)


def _rng_normal(e, cols):
    blk = 1000
    return pl.pallas_call(
        functools.partial(_rng_body, blk, cols),
        grid=(e // blk,),
        out_specs=pl.BlockSpec((blk, cols), lambda i: (i, 0)),
        out_shape=jax.ShapeDtypeStruct((e, cols), jnp.float32),
    )()


def _edge_body(o2, chunk, nchunk, ew,
               a_hbm, b_hbm, src_hbm, dst_hbm, eps_hbm, out_hbm,
               sidx, didx, rowsa, rowsb, epsv, outv,
               insem0, insem1, outsem0, outsem1):
    out = o2 // 2
    wid = lax.axis_index("s") * NC + lax.axis_index("c")
    base = wid * ew
    insem = (insem0, insem1)
    outsem = (outsem0, outsem1)

    pltpu.sync_copy(src_hbm.at[wid], sidx)
    pltpu.sync_copy(dst_hbm.at[wid], didx)

    def stage_in(g, b):
        off = base + g * chunk
        pltpu.async_copy(a_hbm.at[sidx.at[g]], rowsa.at[b], insem[b])
        pltpu.async_copy(b_hbm.at[didx.at[g]], rowsb.at[b], insem[b])
        pltpu.async_copy(eps_hbm.at[pl.ds(off, chunk)], epsv.at[b], insem[b])

    def drain_in(g, b):
        pltpu.make_async_copy(a_hbm.at[sidx.at[g]], rowsa.at[b], insem[b]).wait()
        pltpu.make_async_copy(b_hbm.at[didx.at[g]], rowsb.at[b], insem[b]).wait()
        pltpu.make_async_copy(
            eps_hbm.at[pl.ds(0, chunk)], epsv.at[b], insem[b]).wait()

    def compute(b):
        ngrp = out // LANES

        def row(r, c):
            lo = [pl.ds(j * LANES, LANES) for j in range(ngrp)]
            hi = [pl.ds(out + j * LANES, LANES) for j in range(ngrp)]
            els = [jnp.exp(rowsa[b, r, hi[j]] + rowsb[b, r, hi[j]])
                   for j in range(ngrp)]
            mus = [rowsa[b, r, lo[j]] + rowsb[b, r, lo[j]]
                   for j in range(ngrp)]
            enz = [jnp.exp(-(mus[j] + els[j] * epsv[b, r, lo[j]]))
                   for j in range(ngrp)]
            for j in range(ngrp):
                outv[b, r, lo[j]] = 1.0 / (1.0 + enz[j])
            return c
        lax.fori_loop(0, chunk, row, 0, unroll=2)

    def issue_out(g, b):
        off = base + g * chunk
        pltpu.async_copy(outv.at[b], out_hbm.at[pl.ds(off, chunk)], outsem[b])

    def drain_out(b):
        pltpu.make_async_copy(
            outv.at[b], out_hbm.at[pl.ds(0, chunk)], outsem[b]).wait()

    stage_in(0, 0)
    stage_in(1, 1)

    def pair(p, carry):
        for b in range(2):
            g = 2 * p + b
            drain_in(g, b)

            @pl.when(p > 0)
            def _():
                drain_out(b)

            compute(b)
            issue_out(g, b)

            @pl.when(g + 2 < nchunk)
            def _():
                stage_in(g + 2, b)
        return carry

    lax.fori_loop(0, nchunk // 2, pair, 0)
    drain_out(0)
    drain_out(1)


def _edge_sample(A, B, src, dst, eps):
    n, o2 = A.shape
    e = eps.shape[0]
    out = o2 // 2
    ew = e // NW
    chunk = 40
    nchunk = ew // chunk
    mesh = plsc.VectorSubcoreMesh(core_axis_name="c", subcore_axis_name="s")
    kern = pl.kernel(
        functools.partial(_edge_body, o2, chunk, nchunk, ew),
        mesh=mesh,
        out_type=jax.ShapeDtypeStruct((e, out), jnp.float32),
        scratch_types=[
            pltpu.VMEM((nchunk, chunk), jnp.int32),
            pltpu.VMEM((nchunk, chunk), jnp.int32),
            pltpu.VMEM((2, chunk, o2), jnp.float32),
            pltpu.VMEM((2, chunk, o2), jnp.float32),
            pltpu.VMEM((2, chunk, out), jnp.float32),
            pltpu.VMEM((2, chunk, out), jnp.float32),
            pltpu.SemaphoreType.DMA,
            pltpu.SemaphoreType.DMA,
            pltpu.SemaphoreType.DMA,
            pltpu.SemaphoreType.DMA,
        ],
    )
    return kern(A, B, src.reshape(NW, nchunk, chunk),
                dst.reshape(NW, nchunk, chunk), eps)


def kernel(h, edge_index, W_src, W_dst, W_fc):
    e = edge_index.shape[1]
    out = W_fc.shape[0] // 2
    A, B = _project(h, W_src, W_dst, W_fc)
    src = edge_index[0].astype(jnp.int32)
    dst = edge_index[1].astype(jnp.int32)
    eps = _rng_normal(e, out)
    return _edge_sample(A, B, src, dst, eps)


# speedup vs baseline: 1.1456x; 1.1456x over previous
import functools

import jax
import jax.numpy as jnp
from jax import lax
from jax.experimental import pallas as pl
from jax.experimental.pallas import tpu as pltpu
from jax.experimental.pallas import tpu_sc as plsc

NC = 2
NS = 16
NW = NC * NS
LANES = 16


def _proj_body(h_ref, ws_ref, wd_ref, wf_ref, a_ref, b_ref):
    h = h_ref[...]
    wf = wf_ref[...]
    dn = (((1,), (1,)), ((), ()))
    ts = lax.dot_general(h, ws_ref[...], dn, preferred_element_type=jnp.float32)
    a_ref[...] = lax.dot_general(ts, wf, dn, preferred_element_type=jnp.float32)
    td = lax.dot_general(h, wd_ref[...], dn, preferred_element_type=jnp.float32)
    b_ref[...] = lax.dot_general(td, wf, dn, preferred_element_type=jnp.float32)


def _project(h, W_src, W_dst, W_fc):
    n, f = h.shape
    o2 = W_src.shape[0]
    blk = 2000 if n % 2000 == 0 else n
    grid = (n // blk,)
    return pl.pallas_call(
        _proj_body,
        grid=grid,
        in_specs=[
            pl.BlockSpec((blk, f), lambda i: (i, 0)),
            pl.BlockSpec((o2, f), lambda i: (0, 0)),
            pl.BlockSpec((o2, f), lambda i: (0, 0)),
            pl.BlockSpec((o2, o2), lambda i: (0, 0)),
        ],
        out_specs=[
            pl.BlockSpec((blk, o2), lambda i: (i, 0)),
            pl.BlockSpec((blk, o2), lambda i: (i, 0)),
        ],
        out_shape=[
            jax.ShapeDtypeStruct((n, o2), jnp.float32),
            jax.ShapeDtypeStruct((n, o2), jnp.float32),
        ],
    )(h, W_src, W_dst, W_fc)


_TF_ROT = ((13, 15, 26, 6), (17, 29, 16, 24))


def _rotl(x, r):
    return (x << jnp.uint32(r)) | (x >> jnp.uint32(32 - r))


def _threefry_bits(idx):
    k1 = jnp.uint32(0)
    k2 = jnp.uint32(42)
    ks = (k1, k2, k1 ^ k2 ^ jnp.uint32(0x1BD11BDA))
    x0 = jnp.zeros_like(idx) + ks[0]
    x1 = idx + ks[1]
    for i in range(5):
        for r in _TF_ROT[i % 2]:
            x0 = x0 + x1
            x1 = _rotl(x1, r)
            x1 = x1 ^ x0
        x0 = x0 + ks[(i + 1) % 3]
        x1 = x1 + ks[(i + 2) % 3] + jnp.uint32(i + 1)
    return x0 ^ x1


def _erfinv(x):
    w = -jnp.log1p(-x * x)
    wl = w - 2.5
    p = jnp.float32(2.81022636e-08)
    for c in (3.43273939e-07, -3.5233877e-06, -4.39150654e-06, 0.00021858087,
              -0.00125372503, -0.00417768164, 0.246640727, 1.50140941):
        p = jnp.float32(c) + p * wl
    ws = jnp.sqrt(w) - 3.0
    q = jnp.float32(-0.000200214257)
    for c in (0.000100950558, 0.00134934322, -0.00367342844, 0.00573950773,
              -0.0076224613, 0.00943887047, 1.00167406, 2.83297682):
        q = jnp.float32(c) + q * ws
    return jnp.where(w < 5.0, p, q) * x


def _rng_body(row0, blk, cols, o_ref):
    i = pl.program_id(0)
    base = (row0 + i * blk) * cols
    r_iota = lax.broadcasted_iota(jnp.int32, (blk, cols), 0)
    c_iota = lax.broadcasted_iota(jnp.int32, (blk, cols), 1)
    idx = (base + r_iota * cols + c_iota).astype(jnp.uint32)
    bits = _threefry_bits(idx)
    f = lax.bitcast_convert_type((bits >> jnp.uint32(9)) | jnp.uint32(0x3F800000),
                                 jnp.float32)
    lo = jnp.float32(-0.99999994)
    hi = jnp.float32(1.0)
    u = jnp.maximum(lo, f * (hi - lo) + (lo - (hi - lo)))
    o_ref[...] = jnp.float32(1.4142135623730951) * _erfinv(u)


def _rng_normal(row0, rows, cols):
    blk = 640
    return pl.pallas_call(
        functools.partial(_rng_body, row0, blk, cols),
        grid=(rows // blk,),
        out_specs=pl.BlockSpec((blk, cols), lambda i: (i, 0)),
        out_shape=jax.ShapeDtypeStruct((rows, cols), jnp.float32),
    )()


def _edge_body(o2, chunk, nchunk, ew,
               a_hbm, b_hbm, src_hbm, dst_hbm, eps_hbm, out_hbm,
               sidx, didx, rowsa, rowsb, epsv, outv,
               insem0, insem1, outsem0, outsem1):
    out = o2 // 2
    wid = lax.axis_index("s") * NC + lax.axis_index("c")
    base = wid * ew
    insem = (insem0, insem1)
    outsem = (outsem0, outsem1)

    pltpu.sync_copy(src_hbm.at[wid], sidx)
    pltpu.sync_copy(dst_hbm.at[wid], didx)

    def stage_in(g, b):
        off = base + g * chunk
        pltpu.async_copy(a_hbm.at[sidx.at[g]], rowsa.at[b], insem[b])
        pltpu.async_copy(b_hbm.at[didx.at[g]], rowsb.at[b], insem[b])
        pltpu.async_copy(eps_hbm.at[pl.ds(off, chunk)], epsv.at[b], insem[b])

    def drain_in(g, b):
        pltpu.make_async_copy(a_hbm.at[sidx.at[g]], rowsa.at[b], insem[b]).wait()
        pltpu.make_async_copy(b_hbm.at[didx.at[g]], rowsb.at[b], insem[b]).wait()
        pltpu.make_async_copy(
            eps_hbm.at[pl.ds(0, chunk)], epsv.at[b], insem[b]).wait()

    def compute(b):
        ngrp = out // LANES

        def row(r, c):
            lo = [pl.ds(j * LANES, LANES) for j in range(ngrp)]
            hi = [pl.ds(out + j * LANES, LANES) for j in range(ngrp)]
            els = [jnp.exp(rowsa[b, r, hi[j]] + rowsb[b, r, hi[j]])
                   for j in range(ngrp)]
            mus = [rowsa[b, r, lo[j]] + rowsb[b, r, lo[j]]
                   for j in range(ngrp)]
            enz = [jnp.exp(-(mus[j] + els[j] * epsv[b, r, lo[j]]))
                   for j in range(ngrp)]
            for j in range(ngrp):
                outv[b, r, lo[j]] = 1.0 / (1.0 + enz[j])
            return c
        lax.fori_loop(0, chunk, row, 0, unroll=2)

    def issue_out(g, b):
        off = base + g * chunk
        pltpu.async_copy(outv.at[b], out_hbm.at[pl.ds(off, chunk)], outsem[b])

    def drain_out(b):
        pltpu.make_async_copy(
            outv.at[b], out_hbm.at[pl.ds(0, chunk)], outsem[b]).wait()

    stage_in(0, 0)
    stage_in(1, 1)

    def pair(p, carry):
        for b in range(2):
            g = 2 * p + b
            drain_in(g, b)

            @pl.when(p > 0)
            def _():
                drain_out(b)

            compute(b)
            issue_out(g, b)

            @pl.when(g + 2 < nchunk)
            def _():
                stage_in(g + 2, b)
        return carry

    lax.fori_loop(0, nchunk // 2, pair, 0)
    drain_out(0)
    drain_out(1)


def _edge_sample(A, B, src, dst, eps):
    n, o2 = A.shape
    e = eps.shape[0]
    out = o2 // 2
    ew = e // NW
    chunk = 40
    nchunk = ew // chunk
    mesh = plsc.VectorSubcoreMesh(core_axis_name="c", subcore_axis_name="s")
    kern = pl.kernel(
        functools.partial(_edge_body, o2, chunk, nchunk, ew),
        mesh=mesh,
        out_type=jax.ShapeDtypeStruct((e, out), jnp.float32),
        scratch_types=[
            pltpu.VMEM((nchunk, chunk), jnp.int32),
            pltpu.VMEM((nchunk, chunk), jnp.int32),
            pltpu.VMEM((2, chunk, o2), jnp.float32),
            pltpu.VMEM((2, chunk, o2), jnp.float32),
            pltpu.VMEM((2, chunk, out), jnp.float32),
            pltpu.VMEM((2, chunk, out), jnp.float32),
            pltpu.SemaphoreType.DMA,
            pltpu.SemaphoreType.DMA,
            pltpu.SemaphoreType.DMA,
            pltpu.SemaphoreType.DMA,
        ],
    )
    return kern(A, B, src.reshape(NW, nchunk, chunk),
                dst.reshape(NW, nchunk, chunk), eps)


def kernel(h, edge_index, W_src, W_dst, W_fc):
    e = edge_index.shape[1]
    out = W_fc.shape[0] // 2
    A, B = _project(h, W_src, W_dst, W_fc)
    src = edge_index[0].astype(jnp.int32)
    dst = edge_index[1].astype(jnp.int32)
    sizes = [81920, 81920, 81920, e - 3 * 81920] if e == 320000 else [e]
    pieces = []
    row0 = 0
    for sz in sizes:
        eps_k = _rng_normal(row0, sz, out)
        pieces.append(_edge_sample(A, B, src[row0:row0 + sz],
                                   dst[row0:row0 + sz], eps_k))
        row0 += sz
    if len(pieces) == 1:
        return pieces[0]
    return jnp.concatenate(pieces, axis=0)


# speedup vs baseline: 1.2550x; 1.0954x over previous
import functools

import jax
import jax.numpy as jnp
from jax import lax
from jax.experimental import pallas as pl
from jax.experimental.pallas import tpu as pltpu
from jax.experimental.pallas import tpu_sc as plsc

NC = 2
NS = 16
NW = NC * NS
LANES = 16


def _proj_body(h_ref, ws_ref, wd_ref, wf_ref, a_ref, b_ref):
    h = h_ref[...]
    wf = wf_ref[...]
    dn = (((1,), (1,)), ((), ()))
    ts = lax.dot_general(h, ws_ref[...], dn, preferred_element_type=jnp.float32)
    a_ref[...] = lax.dot_general(ts, wf, dn, preferred_element_type=jnp.float32)
    td = lax.dot_general(h, wd_ref[...], dn, preferred_element_type=jnp.float32)
    b_ref[...] = lax.dot_general(td, wf, dn, preferred_element_type=jnp.float32)


def _project(h, W_src, W_dst, W_fc):
    n, f = h.shape
    o2 = W_src.shape[0]
    blk = 2000 if n % 2000 == 0 else n
    grid = (n // blk,)
    return pl.pallas_call(
        _proj_body,
        grid=grid,
        in_specs=[
            pl.BlockSpec((blk, f), lambda i: (i, 0)),
            pl.BlockSpec((o2, f), lambda i: (0, 0)),
            pl.BlockSpec((o2, f), lambda i: (0, 0)),
            pl.BlockSpec((o2, o2), lambda i: (0, 0)),
        ],
        out_specs=[
            pl.BlockSpec((blk, o2), lambda i: (i, 0)),
            pl.BlockSpec((blk, o2), lambda i: (i, 0)),
        ],
        out_shape=[
            jax.ShapeDtypeStruct((n, o2), jnp.float32),
            jax.ShapeDtypeStruct((n, o2), jnp.float32),
        ],
    )(h, W_src, W_dst, W_fc)


_TF_ROT = ((13, 15, 26, 6), (17, 29, 16, 24))


def _rotl(x, r):
    return (x << jnp.uint32(r)) | (x >> jnp.uint32(32 - r))


def _threefry_bits(idx):
    k1 = jnp.uint32(0)
    k2 = jnp.uint32(42)
    ks = (k1, k2, k1 ^ k2 ^ jnp.uint32(0x1BD11BDA))
    x0 = jnp.zeros_like(idx) + ks[0]
    x1 = idx + ks[1]
    for i in range(5):
        for r in _TF_ROT[i % 2]:
            x0 = x0 + x1
            x1 = _rotl(x1, r)
            x1 = x1 ^ x0
        x0 = x0 + ks[(i + 1) % 3]
        x1 = x1 + ks[(i + 2) % 3] + jnp.uint32(i + 1)
    return x0 ^ x1


def _erfinv(x):
    w = -jnp.log1p(-x * x)
    wl = w - 2.5
    p = jnp.float32(2.81022636e-08)
    for c in (3.43273939e-07, -3.5233877e-06, -4.39150654e-06, 0.00021858087,
              -0.00125372503, -0.00417768164, 0.246640727, 1.50140941):
        p = jnp.float32(c) + p * wl
    ws = jnp.sqrt(w) - 3.0
    q = jnp.float32(-0.000200214257)
    for c in (0.000100950558, 0.00134934322, -0.00367342844, 0.00573950773,
              -0.0076224613, 0.00943887047, 1.00167406, 2.83297682):
        q = jnp.float32(c) + q * ws
    return jnp.where(w < 5.0, p, q) * x


def _rng_body(row0, blk, cols, o_ref):
    i = pl.program_id(0)
    base = (row0 + i * blk) * cols
    r_iota = lax.broadcasted_iota(jnp.int32, (blk, cols), 0)
    c_iota = lax.broadcasted_iota(jnp.int32, (blk, cols), 1)
    idx = (base + r_iota * cols + c_iota).astype(jnp.uint32)
    bits = _threefry_bits(idx)
    f = lax.bitcast_convert_type((bits >> jnp.uint32(9)) | jnp.uint32(0x3F800000),
                                 jnp.float32)
    lo = jnp.float32(-0.99999994)
    hi = jnp.float32(1.0)
    u = jnp.maximum(lo, f * (hi - lo) + (lo - (hi - lo)))
    o_ref[...] = jnp.float32(1.4142135623730951) * _erfinv(u)


def _rng_normal(row0, rows, cols):
    blk = 640
    return pl.pallas_call(
        functools.partial(_rng_body, row0, blk, cols),
        grid=(rows // blk,),
        out_specs=pl.BlockSpec((blk, cols), lambda i: (i, 0)),
        out_shape=jax.ShapeDtypeStruct((rows, cols), jnp.float32),
    )()


def _edge_body(o2, chunk, nchunk, ew,
               a_hbm, b_hbm, src_hbm, dst_hbm, eps_hbm, out_hbm,
               sidx, didx, rowsa, rowsb, epsv, outv,
               insem0, insem1, outsem0, outsem1):
    out = o2 // 2
    wid = lax.axis_index("s") * NC + lax.axis_index("c")
    base = wid * ew
    insem = (insem0, insem1)
    outsem = (outsem0, outsem1)

    pltpu.sync_copy(src_hbm.at[wid], sidx)
    pltpu.sync_copy(dst_hbm.at[wid], didx)

    def stage_in(g, b):
        off = base + g * chunk
        pltpu.async_copy(a_hbm.at[sidx.at[g]], rowsa.at[b], insem[b])
        pltpu.async_copy(b_hbm.at[didx.at[g]], rowsb.at[b], insem[b])
        pltpu.async_copy(eps_hbm.at[pl.ds(off, chunk)], epsv.at[b], insem[b])

    def drain_in(g, b):
        pltpu.make_async_copy(a_hbm.at[sidx.at[g]], rowsa.at[b], insem[b]).wait()
        pltpu.make_async_copy(b_hbm.at[didx.at[g]], rowsb.at[b], insem[b]).wait()
        pltpu.make_async_copy(
            eps_hbm.at[pl.ds(0, chunk)], epsv.at[b], insem[b]).wait()

    def compute(b):
        ngrp = out // LANES

        def row(r, c):
            lo = [pl.ds(j * LANES, LANES) for j in range(ngrp)]
            hi = [pl.ds(out + j * LANES, LANES) for j in range(ngrp)]
            els = [jnp.exp(rowsa[b, r, hi[j]] + rowsb[b, r, hi[j]])
                   for j in range(ngrp)]
            mus = [rowsa[b, r, lo[j]] + rowsb[b, r, lo[j]]
                   for j in range(ngrp)]
            enz = [jnp.exp(-(mus[j] + els[j] * epsv[b, r, lo[j]]))
                   for j in range(ngrp)]
            for j in range(ngrp):
                outv[b, r, lo[j]] = 1.0 / (1.0 + enz[j])
            return c
        lax.fori_loop(0, chunk, row, 0, unroll=2)

    def issue_out(g, b):
        off = base + g * chunk
        pltpu.async_copy(outv.at[b], out_hbm.at[pl.ds(off, chunk)], outsem[b])

    def drain_out(b):
        pltpu.make_async_copy(
            outv.at[b], out_hbm.at[pl.ds(0, chunk)], outsem[b]).wait()

    stage_in(0, 0)
    stage_in(1, 1)

    def pair(p, carry):
        for b in range(2):
            g = 2 * p + b
            drain_in(g, b)

            @pl.when(p > 0)
            def _():
                drain_out(b)

            compute(b)
            issue_out(g, b)

            @pl.when(g + 2 < nchunk)
            def _():
                stage_in(g + 2, b)
        return carry

    lax.fori_loop(0, nchunk // 2, pair, 0)
    drain_out(0)
    drain_out(1)


def _edge_sample(A, B, src, dst, eps):
    n, o2 = A.shape
    e = eps.shape[0]
    out = o2 // 2
    ew = e // NW
    chunk = 40
    nchunk = ew // chunk
    mesh = plsc.VectorSubcoreMesh(core_axis_name="c", subcore_axis_name="s")
    kern = pl.kernel(
        functools.partial(_edge_body, o2, chunk, nchunk, ew),
        mesh=mesh,
        out_type=jax.ShapeDtypeStruct((e, out), jnp.float32),
        scratch_types=[
            pltpu.VMEM((nchunk, chunk), jnp.int32),
            pltpu.VMEM((nchunk, chunk), jnp.int32),
            pltpu.VMEM((2, chunk, o2), jnp.float32),
            pltpu.VMEM((2, chunk, o2), jnp.float32),
            pltpu.VMEM((2, chunk, out), jnp.float32),
            pltpu.VMEM((2, chunk, out), jnp.float32),
            pltpu.SemaphoreType.DMA,
            pltpu.SemaphoreType.DMA,
            pltpu.SemaphoreType.DMA,
            pltpu.SemaphoreType.DMA,
        ],
    )
    return kern(A, B, src.reshape(NW, nchunk, chunk),
                dst.reshape(NW, nchunk, chunk), eps)


def kernel(h, edge_index, W_src, W_dst, W_fc):
    e = edge_index.shape[1]
    out = W_fc.shape[0] // 2
    A, B = _project(h, W_src, W_dst, W_fc)
    src = edge_index[0].astype(jnp.int32)
    dst = edge_index[1].astype(jnp.int32)
    sizes = ([143360, 81920, 48640, 28160, 17920] if e == 320000 else [e])
    pieces = []
    row0 = 0
    for sz in sizes:
        eps_k = _rng_normal(row0, sz, out)
        pieces.append(_edge_sample(A, B, src[row0:row0 + sz],
                                   dst[row0:row0 + sz], eps_k))
        row0 += sz
    if len(pieces) == 1:
        return pieces[0]
    return jnp.concatenate(pieces, axis=0)
